# baseline (speedup 1.0000x reference)
import functools

import jax
import jax.numpy as jnp
from jax import lax
from jax.experimental import pallas as pl
from jax.experimental.pallas import tpu as pltpu
from jax.experimental.pallas import tpu_sc as plsc

VOCAB = 100000
DIM = 64
B = 16384
L = 50

NC = 2
NS = 16
NW = NC * NS
CH = 128
TCB = B // CH
KPW = TCB // NW
NCH = L * KPW


def _make_kernel():
  mesh = plsc.VectorSubcoreMesh(core_axis_name="c", subcore_axis_name="s")

  @functools.partial(
      pl.kernel,
      mesh=mesh,
      compiler_params=pltpu.CompilerParams(
          use_tc_tiling_on_sc=False, needs_layout_passes=False),
      out_type=jax.ShapeDtypeStruct((L * 8, TCB, 8 * CH), jnp.float32),
      scratch_types=[
          pltpu.VMEM((L, KPW * CH), jnp.int32),
          pltpu.VMEM((CH, DIM), jnp.float32),
          pltpu.VMEM((CH, DIM), jnp.float32),
          pltpu.VMEM((8, 8 * CH), jnp.float32),
          pltpu.VMEM((8, 8 * CH), jnp.float32),
          pltpu.SemaphoreType.DMA,
          pltpu.SemaphoreType.DMA,
      ],
  )
  def emb(table_hbm, xt_hbm, out_hbm, idx_v, g0, g1, t0, t1, gsem, wsem):
    gbufs = (g0, g1)
    tbufs = (t0, t1)
    wid = lax.axis_index("s") * NC + lax.axis_index("c")
    bcol0 = wid * (KPW * CH)

    pltpu.sync_copy(xt_hbm.at[:, pl.ds(bcol0, KPW * CH)], idx_v)

    lanes = lax.iota(jnp.int32, 16)
    bidx = [lanes + g * 16 for g in range(8)]

    def idx_slice(j):
      l = j // KPW
      k = lax.rem(j, KPW)
      return idx_v.at[l, pl.ds(k * CH, CH)]

    def transpose(gbuf, tbuf):
      def dblk(tr, carry):
        for di in range(8):
          d = tr * 8 + di
          dcol = jnp.full((16,), d, jnp.int32)
          for g in range(8):
            val = plsc.load_gather(gbuf, [bidx[g], dcol])
            tbuf[tr, pl.ds(di * CH + g * 16, 16)] = val
        return carry

      lax.fori_loop(0, 8, dblk, 0)

    for u in range(2):
      pltpu.async_copy(table_hbm.at[idx_slice(u)], gbufs[u], gsem)

    def chunk(j, gbuf, tbuf):
      l = j // KPW
      k = lax.rem(j, KPW)
      tcg = wid * KPW + k
      pltpu.make_async_copy(table_hbm.at[idx_slice(j)], gbuf, gsem).wait()

      @pl.when(j >= 2)
      def _():
        pltpu.make_async_copy(tbuf, out_hbm.at[pl.ds(0, 8), 0], wsem).wait()

      transpose(gbuf, tbuf)
      pltpu.async_copy(tbuf, out_hbm.at[pl.ds(l * 8, 8), tcg], wsem)

      @pl.when(j + 2 < NCH)
      def _():
        pltpu.async_copy(table_hbm.at[idx_slice(j + 2)], gbuf, gsem)

    def body(gr, carry):
      for u in range(2):
        chunk(gr * 2 + u, gbufs[u], tbufs[u])
      return carry

    lax.fori_loop(0, NCH // 2, body, 0)

    for u in range(2):
      pltpu.make_async_copy(tbufs[u], out_hbm.at[pl.ds(0, 8), 0], wsem).wait()

  return emb


_emb = _make_kernel()


@jax.jit
def kernel(x, table):
  xt = x.T.astype(jnp.int32)
  q = _emb(table, xt)
  q5 = q.reshape(L, 8, TCB, 8, CH)
  return q5.transpose(2, 4, 0, 1, 3).reshape(B, L, DIM)


# speedup vs baseline: 1.2748x; 1.2748x over previous
import functools

import jax
import jax.numpy as jnp
from jax import lax
from jax.experimental import pallas as pl
from jax.experimental.pallas import tpu as pltpu
from jax.experimental.pallas import tpu_sc as plsc

VOCAB = 100000
DIM = 64
B = 16384
L = 50

NC = 2
NS = 16
NW = NC * NS
CH = 128
TCB = B // CH
KPW = TCB // NW
NCH = L * KPW


def _make_kernel():
  mesh = plsc.VectorSubcoreMesh(core_axis_name="c", subcore_axis_name="s")

  @functools.partial(
      pl.kernel,
      mesh=mesh,
      compiler_params=pltpu.CompilerParams(
          use_tc_tiling_on_sc=False, needs_layout_passes=False),
      out_type=jax.ShapeDtypeStruct((L * 8, TCB, 8 * CH), jnp.float32),
      scratch_types=[
          pltpu.VMEM((L, KPW * CH), jnp.int32),
          pltpu.VMEM((CH, DIM), jnp.float32),
          pltpu.VMEM((CH, DIM), jnp.float32),
          pltpu.VMEM((8, 8 * CH), jnp.float32),
          pltpu.VMEM((8, 8 * CH), jnp.float32),
          pltpu.SemaphoreType.DMA,
          pltpu.SemaphoreType.DMA,
      ],
  )
  def emb(table_hbm, xt_hbm, out_hbm, idx_v, g0, g1, t0, t1, gsem, wsem):
    gbufs = (g0, g1)
    tbufs = (t0, t1)
    wid = lax.axis_index("s") * NC + lax.axis_index("c")
    bcol0 = wid * (KPW * CH)

    pltpu.sync_copy(xt_hbm.at[:, pl.ds(bcol0, KPW * CH)], idx_v)

    lanes = lax.iota(jnp.int32, 16)
    bidx = [lanes + g * 16 for g in range(8)]

    def idx_slice(j):
      l = j // KPW
      k = lax.rem(j, KPW)
      return idx_v.at[l, pl.ds(k * CH, CH)]

    def transpose(gbuf, tbuf):
      def dblk(tr, carry):
        d0 = tr * 8
        for dh in range(4):
          vals = []
          for di in (2 * dh, 2 * dh + 1):
            dcol = jnp.full((16,), 0, jnp.int32) + (d0 + di)
            for g in range(8):
              vals.append(plsc.load_gather(gbuf, [bidx[g], dcol]))
          i = 0
          for di in (2 * dh, 2 * dh + 1):
            for g in range(8):
              tbuf[tr, pl.ds(di * CH + g * 16, 16)] = vals[i]
              i += 1
        return carry

      lax.fori_loop(0, 8, dblk, 0)

    for u in range(2):
      pltpu.async_copy(table_hbm.at[idx_slice(u)], gbufs[u], gsem)

    def chunk(j, gbuf, tbuf):
      l = j // KPW
      k = lax.rem(j, KPW)
      tcg = wid * KPW + k
      pltpu.make_async_copy(table_hbm.at[idx_slice(j)], gbuf, gsem).wait()

      @pl.when(j >= 2)
      def _():
        pltpu.make_async_copy(tbuf, out_hbm.at[pl.ds(0, 8), 0], wsem).wait()

      transpose(gbuf, tbuf)
      pltpu.async_copy(tbuf, out_hbm.at[pl.ds(l * 8, 8), tcg], wsem)

      @pl.when(j + 2 < NCH)
      def _():
        pltpu.async_copy(table_hbm.at[idx_slice(j + 2)], gbuf, gsem)

    def body(gr, carry):
      for u in range(2):
        chunk(gr * 2 + u, gbufs[u], tbufs[u])
      return carry

    lax.fori_loop(0, NCH // 2, body, 0)

    for u in range(2):
      pltpu.make_async_copy(tbufs[u], out_hbm.at[pl.ds(0, 8), 0], wsem).wait()

  return emb


_emb = _make_kernel()


@jax.jit
def kernel(x, table):
  xt = x.T.astype(jnp.int32)
  q = _emb(table, xt)
  q5 = q.reshape(L, 8, TCB, 8, CH)
  return q5.transpose(2, 4, 0, 1, 3).reshape(B, L, DIM)


# speedup vs baseline: 3.6901x; 2.8947x over previous
import functools

import jax
import jax.numpy as jnp
from jax import lax
from jax.experimental import pallas as pl
from jax.experimental.pallas import tpu as pltpu
from jax.experimental.pallas import tpu_sc as plsc

VOCAB = 100000
DIM = 64
B = 16384
L = 50

NC = 2
NS = 16
NW = NC * NS
CH = 128
TCB = B // CH
KPW = TCB // NW
NCH = L * KPW
TP = CH + 1


def _make_kernel():
  mesh = plsc.VectorSubcoreMesh(core_axis_name="c", subcore_axis_name="s")

  @functools.partial(
      pl.kernel,
      mesh=mesh,
      compiler_params=pltpu.CompilerParams(
          use_tc_tiling_on_sc=False, needs_layout_passes=False),
      out_type=jax.ShapeDtypeStruct((L * 8, TCB, 8, CH), jnp.float32),
      scratch_types=[
          pltpu.VMEM((L, KPW * CH), jnp.int32),
          pltpu.VMEM((CH, DIM), jnp.float32),
          pltpu.VMEM((CH, DIM), jnp.float32),
          pltpu.VMEM((DIM, TP), jnp.float32),
          pltpu.VMEM((DIM, TP), jnp.float32),
          pltpu.SemaphoreType.DMA,
          pltpu.SemaphoreType.DMA,
      ],
  )
  def emb(table_hbm, xt_hbm, out_hbm, idx_v, g0, g1, t0, t1, gsem, wsem):
    gbufs = (g0, g1)
    tbufs = (t0, t1)
    wid = lax.axis_index("s") * NC + lax.axis_index("c")
    bcol0 = wid * (KPW * CH)

    pltpu.sync_copy(xt_hbm.at[:, pl.ds(bcol0, KPW * CH)], idx_v)

    lanes = lax.iota(jnp.int32, 16)
    drow = [lanes + q * 16 for q in range(4)]

    def idx_slice(j):
      l = j // KPW
      k = lax.rem(j, KPW)
      return idx_v.at[l, pl.ds(k * CH, CH)]

    def transpose(gbuf, tbuf):
      def brow(it, carry):
        for s in range(4):
          b = it * 4 + s
          bcol = jnp.full((16,), 0, jnp.int32) + b
          vals = [gbuf[b, pl.ds(q * 16, 16)] for q in range(4)]
          for q in range(4):
            plsc.store_scatter(tbuf, [drow[q], bcol], vals[q])
        return carry

      lax.fori_loop(0, CH // 4, brow, 0)

    for u in range(2):
      pltpu.async_copy(table_hbm.at[idx_slice(u)], gbufs[u], gsem)

    def chunk(j, gbuf, tbuf):
      l = j // KPW
      k = lax.rem(j, KPW)
      tcg = wid * KPW + k
      pltpu.make_async_copy(table_hbm.at[idx_slice(j)], gbuf, gsem).wait()

      @pl.when(j >= 2)
      def _():
        for tr in range(8):
          pltpu.make_async_copy(
              tbuf.at[pl.ds(0, 8), pl.ds(0, CH)],
              out_hbm.at[0, 0], wsem).wait()

      transpose(gbuf, tbuf)
      for tr in range(8):
        pltpu.async_copy(
            tbuf.at[pl.ds(tr * 8, 8), pl.ds(0, CH)],
            out_hbm.at[l * 8 + tr, tcg], wsem)

      @pl.when(j + 2 < NCH)
      def _():
        pltpu.async_copy(table_hbm.at[idx_slice(j + 2)], gbuf, gsem)

    def body(gr, carry):
      for u in range(2):
        chunk(gr * 2 + u, gbufs[u], tbufs[u])
      return carry

    lax.fori_loop(0, NCH // 2, body, 0)

    for u in range(2):
      for tr in range(8):
        pltpu.make_async_copy(
            tbufs[u].at[pl.ds(0, 8), pl.ds(0, CH)],
            out_hbm.at[0, 0], wsem).wait()

  return emb


_emb = _make_kernel()


@jax.jit
def kernel(x, table):
  xt = x.T.astype(jnp.int32)
  q = _emb(table, xt)
  q5 = q.reshape(L, 8, TCB, 8, CH)
  return q5.transpose(2, 4, 0, 1, 3).reshape(B, L, DIM)
